# baseline (speedup 1.0000x reference)
import functools

import jax
import jax.numpy as jnp
from jax import lax
from jax.experimental import pallas as pl
from jax.experimental.pallas import tpu as pltpu
from jax.experimental.pallas import tpu_sc as plsc

BATCH = 16384
D_R, D_S, D_C, D_F = 16, 16, 32, 12
V_R, V_S, V_C = 1000, 100000, 1000000
NC, NS, L = 2, 16, 16
RPT = BATCH // NS
NBLK = RPT // L

OFF_R, OFF_S, OFF_C, OFF_F, OFF_B = 0, 16, 32, 64, 76
WLEN = 77 * L


@functools.partial(
    pl.kernel,
    out_type=jax.ShapeDtypeStruct((2 * BATCH,), jnp.float32),
    mesh=plsc.VectorSubcoreMesh(core_axis_name="c", subcore_axis_name="s"),
    compiler_params=pltpu.CompilerParams(
        needs_layout_passes=False, use_tc_tiling_on_sc=True),
    scratch_types=[
        pltpu.VMEM((RPT,), jnp.int32),
        pltpu.VMEM((RPT,), jnp.int32),
        pltpu.VMEM((RPT,), jnp.int32),
        pltpu.VMEM((RPT,), jnp.int32),
        pltpu.VMEM((RPT,), jnp.int32),
        pltpu.VMEM((RPT,), jnp.float32),
        pltpu.VMEM((RPT,), jnp.float32),
        pltpu.VMEM((RPT,), jnp.float32),
        pltpu.VMEM((D_R * V_R,), jnp.float32),
        pltpu.VMEM((D_F * RPT,), jnp.float32),
        pltpu.VMEM((WLEN,), jnp.float32),
        pltpu.SemaphoreType.DMA,
        pltpu.SemaphoreType.DMA,
        pltpu.SemaphoreType.DMA,
    ],
)
def _sc_kernel(region_h, state_h, city_h, feat_t_h, regf_h, statef_h,
               cityf_h, w_h, out_h,
               idx_r, idx_s, idx_c, idxf_a, idxf_b, g_a, g_b, out_v,
               reg_cols, feat_cols, w_v, sem, sem_a, sem_b):
    cid = lax.axis_index("c")
    sid = lax.axis_index("s")
    base = pl.multiple_of(sid * RPT, RPT)

    stage = [
        pltpu.async_copy(region_h.at[pl.ds(base, RPT)], idx_r, sem),
        pltpu.async_copy(state_h.at[pl.ds(base, RPT)], idx_s, sem),
        pltpu.async_copy(city_h.at[pl.ds(base, RPT)], idx_c, sem),
        pltpu.async_copy(w_h, w_v, sem),
        pltpu.async_copy(regf_h, reg_cols, sem),
    ]
    for d in range(D_F):
        stage.append(pltpu.async_copy(
            feat_t_h.at[d, pl.ds(base, RPT)],
            feat_cols.at[pl.ds(d * RPT, RPT)], sem))
    for c in stage:
        c.wait()

    @pl.when(cid == 0)
    def _init0():
        def tile_blk(i, _):
            sl = pl.ds(i * L, L)
            acc = w_v[pl.ds(OFF_B * L, L)]
            for d in range(D_F):
                acc = acc + (feat_cols[pl.ds(d * RPT + i * L, L)]
                             * w_v[pl.ds((OFF_F + d) * L, L)])
            ridx = idx_r[sl]
            for d in range(D_R):
                col = plsc.load_gather(
                    reg_cols, [ridx + jnp.full((L,), d * V_R, jnp.int32)])
                acc = acc + col * w_v[pl.ds((OFF_R + d) * L, L)]
            out_v[sl] = acc
            return 0

        lax.fori_loop(0, NBLK, tile_blk, 0)

    @pl.when(cid == 1)
    def _init1():
        def zero_blk(i, _):
            out_v[pl.ds(i * L, L)] = jnp.zeros((L,), jnp.float32)
            return 0

        lax.fori_loop(0, NBLK, zero_blk, 0)

    def run_table(flat_h, vocab, idx_ref, half, w_off):
        c0 = cid * half

        def build_idx(dst, d):
            def bblk(i, _):
                sl = pl.ds(i * L, L)
                dst[sl] = idx_ref[sl] + d * vocab
                return 0

            lax.fori_loop(0, NBLK, bblk, 0)

        def fire(idxf, g, sem_x):
            pltpu.async_copy(flat_h.at[idxf], g, sem_x)

        def drain(idxf, g, sem_x):
            pltpu.make_async_copy(flat_h.at[idxf], g, sem_x).wait()

        def accumulate(g, d):
            wv = w_v[pl.ds((w_off + d) * L, L)]

            def blk(i, _):
                sl = pl.ds(i * L, L)
                out_v[sl] = out_v[sl] + g[sl] * wv
                return 0

            lax.fori_loop(0, NBLK, blk, 0)

        build_idx(idxf_a, c0)
        fire(idxf_a, g_a, sem_a)

        def pair(k, _):
            d0 = c0 + 2 * k
            build_idx(idxf_b, d0 + 1)
            fire(idxf_b, g_b, sem_b)
            drain(idxf_a, g_a, sem_a)
            accumulate(g_a, d0)

            @pl.when(2 * k + 2 < half)
            def _next_a():
                build_idx(idxf_a, d0 + 2)
                fire(idxf_a, g_a, sem_a)
            drain(idxf_b, g_b, sem_b)
            accumulate(g_b, d0 + 1)
            return 0

        lax.fori_loop(0, half // 2, pair, 0)

    run_table(statef_h, V_S, idx_s, D_S // 2, OFF_S)
    run_table(cityf_h, V_C, idx_c, D_C // 2, OFF_C)

    pltpu.sync_copy(out_v, out_h.at[pl.ds(cid * BATCH + base, RPT)])


def kernel(region, state, city, features, emb_region, emb_state, emb_city,
           W, b):
    w_flat = jnp.repeat(jnp.concatenate([W[0], b]), L)
    region_flat = emb_region.T.reshape(-1)
    state_flat = emb_state.T.reshape(-1)
    city_flat = emb_city.T.reshape(-1)
    partials = _sc_kernel(region.astype(jnp.int32), state.astype(jnp.int32),
                          city.astype(jnp.int32), features.T,
                          region_flat, state_flat, city_flat, w_flat)
    return partials[:BATCH] + partials[BATCH:]


# speedup vs baseline: 11.7900x; 11.7900x over previous
import functools

import jax
import jax.numpy as jnp
from jax import lax
from jax.experimental import pallas as pl
from jax.experimental.pallas import tpu as pltpu
from jax.experimental.pallas import tpu_sc as plsc

BATCH = 16384
D_R, D_S, D_C, D_F = 16, 16, 32, 12
V_R, V_S, V_C = 1000, 100000, 1000000
WIN_S = (V_S // 128) * 128
WIN_C = (V_C // 128) * 128
HWIN = WIN_C // 2
TL_S = V_S - WIN_S
TL_C = V_C - WIN_C
NC, NS, L = 2, 16, 16
RPT = BATCH // NS
CHUNK = 128
NCH = RPT // CHUNK
NBLK = RPT // L

OFF_R, OFF_S, OFF_C, OFF_F, OFF_B = 0, 16, 32, 64, 76
WLEN = 77 * L


@functools.partial(
    pl.kernel,
    out_type=jax.ShapeDtypeStruct((2 * BATCH,), jnp.float32),
    mesh=plsc.VectorSubcoreMesh(core_axis_name="c", subcore_axis_name="s"),
    compiler_params=pltpu.CompilerParams(
        needs_layout_passes=False, use_tc_tiling_on_sc=True),
    scratch_types=[
        pltpu.VMEM((RPT,), jnp.int32),
        pltpu.VMEM((RPT,), jnp.int32),
        pltpu.VMEM((RPT,), jnp.int32),
        pltpu.VMEM((RPT,), jnp.int32),
        pltpu.VMEM((2 * RPT,), jnp.int32),
        pltpu.VMEM((RPT,), jnp.float32),
        pltpu.VMEM((RPT,), jnp.float32),
        pltpu.VMEM((D_R * V_R,), jnp.float32),
        pltpu.VMEM((D_F * RPT,), jnp.float32),
        pltpu.VMEM((D_S * TL_S,), jnp.float32),
        pltpu.VMEM((D_C * TL_C,), jnp.float32),
        pltpu.VMEM((WLEN,), jnp.float32),
        pltpu.VMEM_SHARED((HWIN,), jnp.float32),
        pltpu.VMEM_SHARED((HWIN,), jnp.float32),
        pltpu.SemaphoreType.DMA,
        pltpu.SemaphoreType.DMA,
        pltpu.SemaphoreType.DMA,
    ],
)
def _sc_kernel(region_h, state_h, city_h, feat_t_h, regf_h, statet_h,
               cityt_h, stail_h, ctail_h, w_h, out_h,
               idx_r, idx_s, idx_c, idx_s_cl, idx_c_cl, g_v, out_v,
               reg_cols, feat_cols, stail_v, ctail_v, w_v, col_a, col_b,
               sem, sem_a, sem_b):
    cid = lax.axis_index("c")
    sid = lax.axis_index("s")
    base = pl.multiple_of(sid * RPT, RPT)

    stage = [
        pltpu.async_copy(region_h.at[pl.ds(base, RPT)], idx_r, sem),
        pltpu.async_copy(state_h.at[pl.ds(base, RPT)], idx_s, sem),
        pltpu.async_copy(city_h.at[pl.ds(base, RPT)], idx_c, sem),
        pltpu.async_copy(w_h, w_v, sem),
        pltpu.async_copy(regf_h, reg_cols, sem),
        pltpu.async_copy(stail_h, stail_v, sem),
        pltpu.async_copy(ctail_h, ctail_v, sem),
    ]
    for d in range(D_F):
        stage.append(pltpu.async_copy(
            feat_t_h.at[d, pl.ds(base, RPT)],
            feat_cols.at[pl.ds(d * RPT, RPT)], sem))
    for c in stage:
        c.wait()

    def clamp_blk(i, _):
        sl = pl.ds(i * L, L)
        v_s = idx_s[sl]
        v_c = idx_c[sl]
        idx_s_cl[sl] = jnp.minimum(v_s, WIN_S - 1)
        idx_c_cl[pl.ds(i * L, L)] = jnp.clip(v_c, 0, HWIN - 1)
        idx_c_cl[pl.ds(RPT + i * L, L)] = jnp.clip(v_c - HWIN, 0, HWIN - 1)
        return 0

    lax.fori_loop(0, NBLK, clamp_blk, 0)

    @pl.when(cid == 0)
    def _init0():
        def tile_blk(i, _):
            sl = pl.ds(i * L, L)
            acc = w_v[pl.ds(OFF_B * L, L)]
            for d in range(D_F):
                acc = acc + (feat_cols[pl.ds(d * RPT + i * L, L)]
                             * w_v[pl.ds((OFF_F + d) * L, L)])
            ridx = idx_r[sl]
            for d in range(D_R):
                col = plsc.load_gather(
                    reg_cols, [ridx + jnp.full((L,), d * V_R, jnp.int32)])
                acc = acc + col * w_v[pl.ds((OFF_R + d) * L, L)]
            out_v[sl] = acc
            return 0

        lax.fori_loop(0, NBLK, tile_blk, 0)

    @pl.when(cid == 1)
    def _init1():
        def zero_blk(i, _):
            out_v[pl.ds(i * L, L)] = jnp.zeros((L,), jnp.float32)
            return 0

        lax.fori_loop(0, NBLK, zero_blk, 0)


    def pipeline(nsteps_half, src_slice, gather_compute):
        @pl.when(sid == 0)
        def _prologue():
            src, win = src_slice(0)
            pltpu.async_copy(src, col_a.at[pl.ds(0, win)], sem_a)

        def pair(k, _):
            s0 = 2 * k

            @pl.when(sid == 0)
            def _wait_a():
                src, win = src_slice(s0)
                pltpu.make_async_copy(
                    src, col_a.at[pl.ds(0, win)], sem_a).wait()
            plsc.subcore_barrier()

            @pl.when(sid == 0)
            def _start_b():
                src, win = src_slice(s0 + 1)
                pltpu.async_copy(src, col_b.at[pl.ds(0, win)], sem_b)
            gather_compute(col_a, s0)

            @pl.when(sid == 0)
            def _wait_b():
                src, win = src_slice(s0 + 1)
                pltpu.make_async_copy(
                    src, col_b.at[pl.ds(0, win)], sem_b).wait()
            plsc.subcore_barrier()

            @pl.when(jnp.logical_and(sid == 0, 2 * k + 2 < 2 * nsteps_half))
            def _start_a():
                src, win = src_slice(s0 + 2)
                pltpu.async_copy(src, col_a.at[pl.ds(0, win)], sem_a)
            gather_compute(col_b, s0 + 1)
            return 0

        lax.fori_loop(0, nsteps_half, pair, 0)

    sc0 = cid * (D_S // 2)

    def state_src(s):
        return statet_h.at[sc0 + s, pl.ds(0, WIN_S)], WIN_S

    def state_gc(colbuf, s):
        d = sc0 + s
        cps = [
            pltpu.async_copy(
                colbuf.at[idx_s_cl.at[pl.ds(j * CHUNK, CHUNK)]],
                g_v.at[pl.ds(j * CHUNK, CHUNK)], sem)
            for j in range(NCH)
        ]
        for c in cps:
            c.wait()
        wv = w_v[pl.ds((OFF_S + d) * L, L)]
        tbase = d * TL_S

        def blk(i, _):
            sl = pl.ds(i * L, L)
            v = idx_s[sl]
            tv = plsc.load_gather(
                stail_v, [jnp.maximum(v - WIN_S, 0) + tbase])
            val = jnp.where(v >= WIN_S, tv, g_v[sl])
            out_v[sl] = out_v[sl] + val * wv
            return 0

        lax.fori_loop(0, NBLK, blk, 0)

    pipeline(D_S // 4, state_src, state_gc)

    cc0 = cid * (D_C // 2)

    def city_src(s):
        d = cc0 + s // 2
        h = s % 2
        return cityt_h.at[d, pl.ds(h * HWIN, HWIN)], HWIN

    def city_gc(colbuf, s):
        d = cc0 + s // 2
        h = s % 2
        lo = h * HWIN
        cps = [
            pltpu.async_copy(
                colbuf.at[idx_c_cl.at[pl.ds(h * RPT + j * CHUNK, CHUNK)]],
                g_v.at[pl.ds(j * CHUNK, CHUNK)], sem)
            for j in range(NCH)
        ]
        for c in cps:
            c.wait()
        wv = w_v[pl.ds((OFF_C + d) * L, L)]
        tbase = d * TL_C
        h_f = (h * jnp.ones((), jnp.float32)) * jnp.ones((L,), jnp.float32)

        def blk(i, _):
            sl = pl.ds(i * L, L)
            v = idx_c[sl]
            in_rng = jnp.logical_and(v >= lo, v < lo + HWIN)
            tv = plsc.load_gather(
                ctail_v, [jnp.maximum(v - WIN_C, 0) + tbase])
            zero = jnp.zeros((L,), jnp.float32)
            val = (jnp.where(in_rng, g_v[sl], zero)
                   + jnp.where(v >= WIN_C, tv, zero) * h_f)
            out_v[sl] = out_v[sl] + val * wv
            return 0

        lax.fori_loop(0, NBLK, blk, 0)

    pipeline(D_C // 2, city_src, city_gc)

    pltpu.sync_copy(out_v, out_h.at[pl.ds(cid * BATCH + base, RPT)])


def kernel(region, state, city, features, emb_region, emb_state, emb_city,
           W, b):
    w_flat = jnp.repeat(jnp.concatenate([W[0], b]), L)
    region_flat = emb_region.T.reshape(-1)
    state_tail = emb_state[WIN_S:].T.reshape(-1)
    city_tail = emb_city[WIN_C:].T.reshape(-1)
    partials = _sc_kernel(region.astype(jnp.int32), state.astype(jnp.int32),
                          city.astype(jnp.int32), features.T,
                          region_flat, emb_state.T, emb_city.T,
                          state_tail, city_tail, w_flat)
    return partials[:BATCH] + partials[BATCH:]
